# baseline (speedup 1.0000x reference)
import jax
import jax.numpy as jnp
from jax import lax
from jax.experimental import pallas as pl
from jax.experimental.pallas import tpu as pltpu
from jax.experimental.pallas import tpu_sc as plsc

_BUFF = 10
_B, _N, _H, _W = 16, 512, 512, 512
_NC, _NS, _L = 2, 16, 16
_NW = _NC * _NS
_PT = (_B * _N) // _NW
_STEPS = _PT // _L
_ROWS = _PT * 4
_IMG = _H * _W


def _integral_body(gt_ref, ps_ref, pts_ref, sco_ref, e_ref, pxy_ref, s1_ref):
    m = (gt_ref[0, 0] == ps_ref[0]).astype(jnp.float32)
    row = lax.broadcasted_iota(jnp.int32, (_H, _H), 0)
    col = lax.broadcasted_iota(jnp.int32, (_H, _H), 1)
    ls = (row > col).astype(jnp.float32)
    us = (row < col).astype(jnp.float32)
    a = jnp.dot(ls, m, preferred_element_type=jnp.float32)
    e = jnp.dot(a, us, preferred_element_type=jnp.float32)
    e_ref[...] = e.reshape(_IMG)
    eye = (row == col).astype(jnp.float32)
    pts_t = lax.dot_general(pts_ref[0], eye, (((0,), (0,)), ((), ())),
                            precision=lax.Precision.HIGHEST,
                            preferred_element_type=jnp.float32)
    pxy_ref[...] = pts_t.reshape(2 * _N)
    s1_ref[...] = sco_ref[0, 0]


def _integral_images(gt_segment, ps_segments, points, scores):
    return pl.pallas_call(
        _integral_body,
        grid=(_B,),
        in_specs=[
            pl.BlockSpec((1, 1, _H, _W), lambda b: (b, 0, 0, 0)),
            pl.BlockSpec((1, _H, _W), lambda b: (b, 0, 0)),
            pl.BlockSpec((1, _N, 2), lambda b: (b, 0, 0)),
            pl.BlockSpec((1, 1, _N), lambda b: (b, 0, 0)),
        ],
        out_specs=[
            pl.BlockSpec((_IMG,), lambda b: (b,)),
            pl.BlockSpec((2 * _N,), lambda b: (b,)),
            pl.BlockSpec((_N,), lambda b: (b,)),
        ],
        out_shape=[
            jax.ShapeDtypeStruct((_B * _IMG,), jnp.float32),
            jax.ShapeDtypeStruct((_B * 2 * _N,), jnp.float32),
            jax.ShapeDtypeStruct((_B * _N,), jnp.float32),
        ],
    )(gt_segment, ps_segments, points, scores.reshape(_B, 1, _N))


def _sc_body(pxy_h, sco_h, tab_h, sgt_h, part_h,
             xy_v, sco_v, idx_v, den_v, vals_v, sgt_v, part_v, sem):
    wid = lax.axis_index("s") * _NC + lax.axis_index("c")
    base = wid * _PT

    b = lax.shift_right_logical(wid, 1)
    xoff = b * (2 * _N) + jnp.bitwise_and(wid, 1) * _PT
    pltpu.sync_copy(pxy_h.at[pl.ds(xoff, _PT)], xy_v.at[pl.ds(0, _PT)])
    pltpu.sync_copy(pxy_h.at[pl.ds(xoff + _N, _PT)], xy_v.at[pl.ds(_PT, _PT)])
    pltpu.sync_copy(sco_h.at[pl.ds(base, _PT)], sco_v)

    copies = []
    for i in range(_STEPS):
        x = xy_v[pl.ds(i * _L, _L)]
        y = xy_v[pl.ds(_PT + i * _L, _L)]
        px = ((x + 1.0) * (0.5 * _W)).astype(jnp.int32)
        py = ((y + 1.0) * (0.5 * _H)).astype(jnp.int32)
        tx = jnp.clip(px - _BUFF, 0, _W - 1)
        bx = jnp.clip(px + _BUFF, 0, _W - 1)
        ty = jnp.clip(py - _BUFF, 0, _H - 1)
        by = jnp.clip(py + _BUFF, 0, _H - 1)
        den_v[pl.ds(i * _L, _L)] = ((bx - tx) * (by - ty)).astype(jnp.float32)
        fb = b * _IMG
        yb = by * _W
        yt = ty * _W
        for c, f in enumerate((fb + yb + bx, fb + yt + bx,
                               fb + yb + tx, fb + yt + tx)):
            idx_v[pl.ds(c * _PT + i * _L, _L)] = f
        if i == _STEPS // 2 - 1:
            copies += [
                pltpu.async_copy(tab_h.at[idx_v.at[pl.ds(c * _PT, 128)]],
                                 vals_v.at[pl.ds(c * _PT, 128)], sem)
                for c in range(4)
            ]
    copies += [
        pltpu.async_copy(tab_h.at[idx_v.at[pl.ds(c * _PT + 128, 128)]],
                         vals_v.at[pl.ds(c * _PT + 128, 128)], sem)
        for c in range(4)
    ]
    for c in copies:
        c.wait()

    acc = jnp.zeros((_L,), jnp.float32)
    for i in range(_STEPS):
        v = [vals_v[pl.ds(c * _PT + i * _L, _L)] for c in range(4)]
        cnt = v[0] - v[1] - v[2] + v[3]
        s = jnp.clip(cnt / den_v[pl.ds(i * _L, _L)], 0.0, 1.0)
        sgt_v[pl.ds(i * _L, _L)] = s
        d = sco_v[pl.ds(i * _L, _L)] - s
        acc = acc + d * d

    part_v[...] = acc
    pltpu.sync_copy(sgt_v, sgt_h.at[wid])
    pltpu.sync_copy(part_v, part_h.at[wid])


def _sc_scores_gt(pxy, sco1, table):
    mesh = plsc.VectorSubcoreMesh(core_axis_name="c", subcore_axis_name="s")
    run = pl.kernel(
        _sc_body,
        out_type=[
            jax.ShapeDtypeStruct((_NW, _PT), jnp.float32),
            jax.ShapeDtypeStruct((_NW, _L), jnp.float32),
        ],
        mesh=mesh,
        scratch_types=[
            pltpu.VMEM((_PT * 2,), jnp.float32),
            pltpu.VMEM((_PT,), jnp.float32),
            pltpu.VMEM((_ROWS,), jnp.int32),
            pltpu.VMEM((_PT,), jnp.float32),
            pltpu.VMEM((_ROWS,), jnp.float32),
            pltpu.VMEM((_PT,), jnp.float32),
            pltpu.VMEM((_L,), jnp.float32),
            pltpu.SemaphoreType.DMA,
        ],
    )
    return run(pxy, sco1, table)


def _loss_body(p_ref, out_ref):
    p = p_ref[...]
    out_ref[0, 0] = jnp.sum(p) * (1.0 / (_B * _N))


def _loss(partials):
    out = pl.pallas_call(
        _loss_body,
        out_specs=pl.BlockSpec(memory_space=pltpu.SMEM),
        out_shape=jax.ShapeDtypeStruct((1, 1), jnp.float32),
    )(partials)
    return out[0, 0]


def kernel(scores, points, gt_segment, ps_segments):
    table, pxy, sco1 = _integral_images(gt_segment, ps_segments, points, scores)
    sgt32, part = _sc_scores_gt(pxy, sco1, table)
    return (_loss(part), sgt32.reshape(_B, _N))


# speedup vs baseline: 1.0345x; 1.0345x over previous
import jax
import jax.numpy as jnp
from jax import lax
from jax.experimental import pallas as pl
from jax.experimental.pallas import tpu as pltpu
from jax.experimental.pallas import tpu_sc as plsc

_BUFF = 10
_B, _N, _H, _W = 16, 512, 512, 512
_NC, _NS, _L = 2, 16, 16
_NW = _NC * _NS
_PT = (_B * _N) // _NW
_STEPS = _PT // _L
_ROWS = _PT * 4
_IMG = _H * _W


def _integral_body(gt_ref, ps_ref, pts_ref, e_ref, pxy_ref):
    m = (gt_ref[0, 0] == ps_ref[0]).astype(jnp.float32)
    row = lax.broadcasted_iota(jnp.int32, (_H, _H), 0)
    col = lax.broadcasted_iota(jnp.int32, (_H, _H), 1)
    ls = (row > col).astype(jnp.float32)
    us = (row < col).astype(jnp.float32)
    a = jnp.dot(ls, m, preferred_element_type=jnp.float32)
    e = jnp.dot(a, us, preferred_element_type=jnp.float32)
    e_ref[...] = e.reshape(_IMG)
    eye = (row == col).astype(jnp.float32)
    pts_t = lax.dot_general(pts_ref[0], eye, (((0,), (0,)), ((), ())),
                            precision=lax.Precision.HIGHEST,
                            preferred_element_type=jnp.float32)
    pxy_ref[...] = pts_t.reshape(2 * _N)


def _integral_images(gt_segment, ps_segments, points):
    return pl.pallas_call(
        _integral_body,
        grid=(_B,),
        in_specs=[
            pl.BlockSpec((1, 1, _H, _W), lambda b: (b, 0, 0, 0)),
            pl.BlockSpec((1, _H, _W), lambda b: (b, 0, 0)),
            pl.BlockSpec((1, _N, 2), lambda b: (b, 0, 0)),
        ],
        out_specs=[
            pl.BlockSpec((_IMG,), lambda b: (b,)),
            pl.BlockSpec((2 * _N,), lambda b: (b,)),
        ],
        out_shape=[
            jax.ShapeDtypeStruct((_B * _IMG,), jnp.float32),
            jax.ShapeDtypeStruct((_B * 2 * _N,), jnp.float32),
        ],
    )(gt_segment, ps_segments, points)


def _sc_body(pxy_h, tab_h, sgt_h,
             xy_v, idx_v, den_v, vals_v, sgt_v, sem):
    wid = lax.axis_index("s") * _NC + lax.axis_index("c")
    base = wid * _PT

    b = lax.shift_right_logical(wid, 1)
    xoff = b * (2 * _N) + jnp.bitwise_and(wid, 1) * _PT
    pltpu.sync_copy(pxy_h.at[pl.ds(xoff, _PT)], xy_v.at[pl.ds(0, _PT)])
    pltpu.sync_copy(pxy_h.at[pl.ds(xoff + _N, _PT)], xy_v.at[pl.ds(_PT, _PT)])

    copies = []
    for i in range(_STEPS):
        x = xy_v[pl.ds(i * _L, _L)]
        y = xy_v[pl.ds(_PT + i * _L, _L)]
        px = ((x + 1.0) * (0.5 * _W)).astype(jnp.int32)
        py = ((y + 1.0) * (0.5 * _H)).astype(jnp.int32)
        tx = jnp.clip(px - _BUFF, 0, _W - 1)
        bx = jnp.clip(px + _BUFF, 0, _W - 1)
        ty = jnp.clip(py - _BUFF, 0, _H - 1)
        by = jnp.clip(py + _BUFF, 0, _H - 1)
        den_v[pl.ds(i * _L, _L)] = ((bx - tx) * (by - ty)).astype(jnp.float32)
        fb = b * _IMG
        yb = by * _W
        yt = ty * _W
        for c, f in enumerate((fb + yb + bx, fb + yt + bx,
                               fb + yb + tx, fb + yt + tx)):
            idx_v[pl.ds(c * _PT + i * _L, _L)] = f
        if i == _STEPS // 2 - 1:
            copies += [
                pltpu.async_copy(tab_h.at[idx_v.at[pl.ds(c * _PT, 128)]],
                                 vals_v.at[pl.ds(c * _PT, 128)], sem)
                for c in range(4)
            ]
    copies += [
        pltpu.async_copy(tab_h.at[idx_v.at[pl.ds(c * _PT + 128, 128)]],
                         vals_v.at[pl.ds(c * _PT + 128, 128)], sem)
        for c in range(4)
    ]
    for c in copies:
        c.wait()

    for i in range(_STEPS):
        v = [vals_v[pl.ds(c * _PT + i * _L, _L)] for c in range(4)]
        cnt = v[0] - v[1] - v[2] + v[3]
        s = jnp.clip(cnt / den_v[pl.ds(i * _L, _L)], 0.0, 1.0)
        sgt_v[pl.ds(i * _L, _L)] = s

    pltpu.sync_copy(sgt_v, sgt_h.at[wid])


def _sc_scores_gt(pxy, table):
    mesh = plsc.VectorSubcoreMesh(core_axis_name="c", subcore_axis_name="s")
    run = pl.kernel(
        _sc_body,
        out_type=jax.ShapeDtypeStruct((_NW, _PT), jnp.float32),
        mesh=mesh,
        scratch_types=[
            pltpu.VMEM((_PT * 2,), jnp.float32),
            pltpu.VMEM((_ROWS,), jnp.int32),
            pltpu.VMEM((_PT,), jnp.float32),
            pltpu.VMEM((_ROWS,), jnp.float32),
            pltpu.VMEM((_PT,), jnp.float32),
            pltpu.SemaphoreType.DMA,
        ],
    )
    return run(pxy, table)


def _loss_body(s_ref, g_ref, out_ref):
    s = s_ref[...]
    g = g_ref[...].reshape(_B, 2, _N // 2)
    d0 = g[:, 0, :] - s[:, : _N // 2]
    d1 = g[:, 1, :] - s[:, _N // 2:]
    out_ref[0, 0] = (jnp.sum(d0 * d0) + jnp.sum(d1 * d1)) * (1.0 / (_B * _N))


def _loss(scores, scores_gt32):
    out = pl.pallas_call(
        _loss_body,
        out_specs=pl.BlockSpec(memory_space=pltpu.SMEM),
        out_shape=jax.ShapeDtypeStruct((1, 1), jnp.float32),
    )(scores, scores_gt32)
    return out[0, 0]


def kernel(scores, points, gt_segment, ps_segments):
    table, pxy = _integral_images(gt_segment, ps_segments, points)
    sgt32 = _sc_scores_gt(pxy, table)
    return (_loss(scores, sgt32), sgt32.reshape(_B, _N))
